# baseline (speedup 1.0000x reference)
import jax
import jax.numpy as jnp
from jax.experimental import pallas as pl
from jax.experimental.pallas import tpu as pltpu

_CONF_THRESH = 0.25
_IOU_THRESH = 0.45
_MAX_PER_IMG = 100
_N = 5000
_NC = 80
_R = 40
_L = 128
_NP = _R * _L
_K = 4


_SX1, _SY1, _SX2, _SY2, _SOBJ, _SCCF, _SCPRED, _SAREA = range(8)


def _setup_one(x_ref, scr_ref, k):
    cx = x_ref[k, 0]
    cy = x_ref[k, 1]
    w = x_ref[k, 2]
    h = x_ref[k, 3]
    obj = x_ref[k, 4]

    x1 = cx - w / 2.0
    y1 = cy - h / 2.0
    x2 = cx + w / 2.0
    y2 = cy + h / 2.0
    area = (x2 - x1) * (y2 - y1)

    cls = x_ref[k, 5:5 + _NC]
    ccf = jnp.max(cls, axis=0)
    cidx = jax.lax.broadcasted_iota(jnp.int32, (_NC, _R, _L), 0)
    cpred = jnp.min(jnp.where(cls == ccf[None], cidx, 2**30), axis=0).astype(jnp.float32)

    score = obj * ccf
    ri = jax.lax.broadcasted_iota(jnp.int32, (_R, _L), 0)
    li = jax.lax.broadcasted_iota(jnp.int32, (_R, _L), 1)
    flat = ri * _L + li
    valid = flat < _N

    m0 = jnp.max(jnp.where(valid, score, float("-inf")), keepdims=True)
    conf = jnp.minimum(_CONF_THRESH, m0)
    s0 = jnp.where(valid & (score >= conf), score, -1.0)

    for cp in range(2):
        scr_ref[k, cp, _SX1] = x1
        scr_ref[k, cp, _SY1] = y1
        scr_ref[k, cp, _SX2] = x2
        scr_ref[k, cp, _SY2] = y2
        scr_ref[k, cp, _SOBJ] = obj
        scr_ref[k, cp, _SCCF] = ccf
        scr_ref[k, cp, _SCPRED] = cpred
        scr_ref[k, cp, _SAREA] = area
    return s0


def _comb(va, ia, vb, ib):
    take = (va > vb) | ((va == vb) & (ia <= ib))
    return jnp.where(take, va, vb), jnp.where(take, ia, ib)


def _tpose(a, n):
    eye = jnp.eye(n, dtype=jnp.float32)
    return jax.lax.dot_general(a, eye, (((0,), (0,)), ((), ())),
                               preferred_element_type=jnp.float32)


def _nms_body(x_ref, o_ref, scr_ref):
    s0s = [_setup_one(x_ref, scr_ref, k) for k in range(_K)]

    def step(t, ss):
        cp = jax.lax.rem(t, 2)
        ri = jax.lax.broadcasted_iota(jnp.int32, (_R, _L), 0)
        li = jax.lax.broadcasted_iota(jnp.int32, (_R, _L), 1)
        flatf = (ri * _L + li).astype(jnp.float32)
        sub8 = jax.lax.broadcasted_iota(jnp.int32, (8, _L), 0)
        sub2 = jax.lax.broadcasted_iota(jnp.int32, (2, 1), 0)
        diag8 = (jax.lax.broadcasted_iota(jnp.int32, (8, 8), 0)
                 == jax.lax.broadcasted_iota(jnp.int32, (8, 8), 1))
        ones_mat = jnp.ones((_L, _L), jnp.float32)
        ones_row = jnp.ones((1, _L), jnp.float32)
        out = []
        rows = []
        for k in range(_K):
            s = ss[k]
            v, ix = s[0:8], flatf[0:8]
            for j in range(1, 5):
                v, ix = _comb(v, ix, s[8 * j:8 * j + 8], flatf[8 * j:8 * j + 8])
            v, ix = _tpose(v, 8), _tpose(ix, 8)
            n = 128
            while n > 1:
                n //= 2
                v, ix = _comb(v[:n], ix[:n], v[n:2 * n], ix[n:2 * n])
            v, ix = _tpose(v, 1), _tpose(ix, 1)
            n = 8
            while n > 1:
                n //= 2
                v, ix = _comb(v[:n], ix[:n], v[n:2 * n], ix[n:2 * n])
            vi = jnp.where(sub2 == 0, v, ix)
            bc = jnp.dot(vi, ones_row, preferred_element_type=jnp.float32)
            mrow = bc[0:1, :]
            irow = bc[1:2, :]

            pick = flatf == irow
            pf = pick.astype(jnp.float32)

            sel = jnp.zeros((8, _L), jnp.float32)
            for j in range(8):
                colsum = jnp.sum(pf * scr_ref[k, cp, j], axis=0, keepdims=True)
                sel = sel + jnp.where(sub8 == j, colsum, 0.0)
            bvals = jnp.dot(sel, ones_mat, preferred_element_type=jnp.float32)
            bx1 = bvals[_SX1:_SX1 + 1, :]
            by1 = bvals[_SY1:_SY1 + 1, :]
            bx2 = bvals[_SX2:_SX2 + 1, :]
            by2 = bvals[_SY2:_SY2 + 1, :]
            bcls = bvals[_SCPRED:_SCPRED + 1, :]

            okf = jnp.where(mrow[:, 0:8] >= 0.0, 1.0, 0.0)
            row = jnp.sum(jnp.where(diag8, bvals[:, :8], 0.0), axis=0,
                          keepdims=True) * okf
            rows.append(row)

            x1 = scr_ref[k, cp, _SX1]
            y1 = scr_ref[k, cp, _SY1]
            x2 = scr_ref[k, cp, _SX2]
            y2 = scr_ref[k, cp, _SY2]
            area = scr_ref[k, cp, _SAREA]
            cpred = scr_ref[k, cp, _SCPRED]
            xx1 = jnp.maximum(bx1, x1)
            yy1 = jnp.maximum(by1, y1)
            xx2 = jnp.minimum(bx2, x2)
            yy2 = jnp.minimum(by2, y2)
            inter = jnp.maximum(xx2 - xx1, 0.0) * jnp.maximum(yy2 - yy1, 0.0)
            ba = (bx2 - bx1) * (by2 - by1)
            iou = inter / (ba + area - inter + 1e-9)
            sup = (iou > _IOU_THRESH) & (cpred == bcls)
            out.append(jnp.where(sup | pick, -1.0, s))
        for k in range(_K):
            o_ref[k, pl.ds(t, 1), :] = rows[k]
        return tuple(out)

    jax.lax.fori_loop(0, _MAX_PER_IMG, step, tuple(s0s))


def kernel(x):
    b, n, c = x.shape
    xp = jnp.pad(x, ((0, 0), (0, _NP - n), (0, 0)))
    xt = xp.transpose(0, 2, 1).reshape(b, c, _R, _L)
    out = pl.pallas_call(
        _nms_body,
        grid=(b // _K,),
        in_specs=[pl.BlockSpec((_K, c, _R, _L), lambda i: (i, 0, 0, 0))],
        out_specs=pl.BlockSpec((_K, _MAX_PER_IMG, 8), lambda i: (i, 0, 0)),
        out_shape=jax.ShapeDtypeStruct((b, _MAX_PER_IMG, 8), jnp.float32),
        scratch_shapes=[pltpu.VMEM((_K, 2, 8, _R, _L), jnp.float32)],
        compiler_params=pltpu.CompilerParams(dimension_semantics=("parallel",)),
    )(xt)
    return out[:, :, :7]


# speedup vs baseline: 3.2742x; 3.2742x over previous
import jax
import jax.numpy as jnp
from jax.experimental import pallas as pl
from jax.experimental.pallas import tpu as pltpu

_CONF_THRESH = 0.25
_IOU_THRESH = 0.45
_MAX_PER_IMG = 100
_N = 5000
_NC = 80
_R = 40
_L = 128
_NP = _R * _L
_NEG = float("-inf")
_K = 4


_SX1, _SY1, _SX2, _SY2, _SOBJ, _SCCF, _SCPRED, _SAREA = range(8)


def _setup_one(x_ref, scr_ref, k):
    cx = x_ref[k, 0]
    cy = x_ref[k, 1]
    w = x_ref[k, 2]
    h = x_ref[k, 3]
    obj = x_ref[k, 4]

    x1 = cx - w / 2.0
    y1 = cy - h / 2.0
    x2 = cx + w / 2.0
    y2 = cy + h / 2.0
    area = (x2 - x1) * (y2 - y1)

    cls = x_ref[k, 5:5 + _NC]
    ccf = jnp.max(cls, axis=0)
    cidx = jax.lax.broadcasted_iota(jnp.int32, (_NC, _R, _L), 0)
    cpred = jnp.min(jnp.where(cls == ccf[None], cidx, 2**30), axis=0).astype(jnp.float32)

    score = obj * ccf
    ri = jax.lax.broadcasted_iota(jnp.int32, (_R, _L), 0)
    li = jax.lax.broadcasted_iota(jnp.int32, (_R, _L), 1)
    flat = ri * _L + li
    valid = flat < _N

    m0 = jnp.max(jnp.where(valid, score, _NEG), keepdims=True)
    conf = jnp.minimum(_CONF_THRESH, m0)
    s0 = jnp.where(valid & (score >= conf), score, _NEG)

    for cp in range(2):
        scr_ref[k, cp, _SX1] = x1
        scr_ref[k, cp, _SY1] = y1
        scr_ref[k, cp, _SX2] = x2
        scr_ref[k, cp, _SY2] = y2
        scr_ref[k, cp, _SOBJ] = obj
        scr_ref[k, cp, _SCCF] = ccf
        scr_ref[k, cp, _SCPRED] = cpred
        scr_ref[k, cp, _SAREA] = area
    return s0


def _nms_body(x_ref, o_ref, scr_ref):
    s0s = [_setup_one(x_ref, scr_ref, k) for k in range(_K)]

    def step(t, ss):
        cp = jax.lax.rem(t, 2)
        ri = jax.lax.broadcasted_iota(jnp.int32, (_R, _L), 0)
        li = jax.lax.broadcasted_iota(jnp.int32, (_R, _L), 1)
        flatf = (ri * _L + li).astype(jnp.float32)
        sub8 = jax.lax.broadcasted_iota(jnp.int32, (8, _L), 0)
        diag8 = (jax.lax.broadcasted_iota(jnp.int32, (8, 8), 0)
                 == jax.lax.broadcasted_iota(jnp.int32, (8, 8), 1))
        ones_mat = jnp.ones((_L, _L), jnp.float32)

        ms = [jnp.max(ss[k], keepdims=True) for k in range(_K)]
        iis = [jnp.min(jnp.where(ss[k] == ms[k], flatf, 3.0e7), keepdims=True)
               for k in range(_K)]
        picks = [flatf == iis[k] for k in range(_K)]
        bvalss = []
        for k in range(_K):
            pf = picks[k].astype(jnp.float32)
            sel = jnp.zeros((8, _L), jnp.float32)
            for j in range(8):
                colsum = jnp.sum(pf * scr_ref[k, cp, j], axis=0, keepdims=True)
                sel = sel + jnp.where(sub8 == j, colsum, 0.0)
            bvalss.append(jnp.dot(sel, ones_mat,
                                  preferred_element_type=jnp.float32))
        out = []
        rows = []
        for k in range(_K):
            bvals = bvalss[k]
            bx1 = bvals[_SX1:_SX1 + 1, :]
            by1 = bvals[_SY1:_SY1 + 1, :]
            bx2 = bvals[_SX2:_SX2 + 1, :]
            by2 = bvals[_SY2:_SY2 + 1, :]
            bcls = bvals[_SCPRED:_SCPRED + 1, :]

            okf = jnp.where(ms[k] > _NEG, 1.0, 0.0)
            row = jnp.sum(jnp.where(diag8, bvals[:, :8], 0.0), axis=0,
                          keepdims=True) * okf
            rows.append(row)

            x1 = scr_ref[k, cp, _SX1]
            y1 = scr_ref[k, cp, _SY1]
            x2 = scr_ref[k, cp, _SX2]
            y2 = scr_ref[k, cp, _SY2]
            area = scr_ref[k, cp, _SAREA]
            cpred = scr_ref[k, cp, _SCPRED]
            xx1 = jnp.maximum(bx1, x1)
            yy1 = jnp.maximum(by1, y1)
            xx2 = jnp.minimum(bx2, x2)
            yy2 = jnp.minimum(by2, y2)
            inter = jnp.maximum(xx2 - xx1, 0.0) * jnp.maximum(yy2 - yy1, 0.0)
            ba = (bx2 - bx1) * (by2 - by1)
            iou = inter / (ba + area - inter + 1e-9)
            sup = (iou > _IOU_THRESH) & (cpred == bcls)
            out.append(jnp.where(sup | picks[k], _NEG, ss[k]))
        for k in range(_K):
            o_ref[k, pl.ds(t, 1), :] = rows[k]
        return tuple(out)

    jax.lax.fori_loop(0, _MAX_PER_IMG, step, tuple(s0s))


def kernel(x):
    b, n, c = x.shape
    xp = jnp.pad(x, ((0, 0), (0, _NP - n), (0, 0)))
    xt = xp.transpose(0, 2, 1).reshape(b, c, _R, _L)
    out = pl.pallas_call(
        _nms_body,
        grid=(b // _K,),
        in_specs=[pl.BlockSpec((_K, c, _R, _L), lambda i: (i, 0, 0, 0))],
        out_specs=pl.BlockSpec((_K, _MAX_PER_IMG, 8), lambda i: (i, 0, 0)),
        out_shape=jax.ShapeDtypeStruct((b, _MAX_PER_IMG, 8), jnp.float32),
        scratch_shapes=[pltpu.VMEM((_K, 2, 8, _R, _L), jnp.float32)],
        compiler_params=pltpu.CompilerParams(dimension_semantics=("parallel",)),
    )(xt)
    return out[:, :, :7]
